# baseline (speedup 1.0000x reference)
import functools

import jax
import jax.numpy as jnp
from jax.experimental import pallas as pl
from jax.experimental.pallas import tpu as pltpu

_TOPK = 8
_BLOCK_T = 256
_NEG = -1e30


def _main_kernel(xc_ref, wc_ref, kc_ref, vc_ref, ginter_ref, oc_ref, invk_ref):
    f32 = jnp.float32
    d = kc_ref.shape[1]

    @pl.when(pl.program_id(0) == 0)
    def _():
        kc0 = kc_ref[...]
        k_mag = jnp.sqrt(jnp.sum(kc0 * kc0, axis=0, keepdims=True) + 1e-8)
        invk_ref[...] = 1.0 / k_mag

    xc = xc_ref[...]

    qb = jnp.dot(xc, wc_ref[...], preferred_element_type=f32)

    dot = jnp.dot(qb, kc_ref[...], preferred_element_type=f32)
    u = dot * invk_ref[...]

    q_mag = jnp.sqrt(jnp.sum(qb * qb, axis=1, keepdims=True) + 1e-8)
    invq = 1.0 / q_mag

    m0 = jnp.max(u, axis=1, keepdims=True)
    oh = u == m0
    wd = jnp.where(oh, 1.0, 0.0)
    work = jnp.where(oh, _NEG, u)
    denom = jnp.ones_like(m0)
    for _ in range(_TOPK - 1):
        m = jnp.max(work, axis=1, keepdims=True)
        e = jnp.exp((m - m0) * invq)
        oh = work == m
        wd = jnp.where(oh, e, wd)
        work = jnp.where(oh, _NEG, work)
        denom = denom + e
    wd = wd * (1.0 / denom)

    out_c = jnp.dot(wd, vc_ref[...], preferred_element_type=f32)

    inv_rms = jax.lax.rsqrt(2.0 * jnp.mean(out_c * out_c, axis=1, keepdims=True)
                            + 1e-8)
    oc_ref[...] = out_c * inv_rms * ginter_ref[...]


@functools.partial(jax.jit, static_argnames=())
def kernel(x, keys, values, W_qr, W_qi, gamma):
    b, l, d, _ = x.shape
    s = keys.shape[0]
    t = b * l
    xc = x.reshape(t, 2 * d)
    wc = jnp.stack(
        [jnp.concatenate([W_qr, W_qi], axis=1),
         jnp.concatenate([-W_qi, W_qr], axis=1)],
        axis=1,
    ).reshape(2 * d, 2 * d)
    kc = jnp.concatenate([keys[..., 0].T, keys[..., 1].T], axis=0)
    vc = values.reshape(s, 2 * d)
    ginter = jnp.stack([gamma, gamma], axis=-1).reshape(1, 2 * d)

    bt = min(_BLOCK_T, t)
    grid = (t // bt,)
    full = lambda shape: pl.BlockSpec(shape, lambda i: (0, 0))

    o_c = pl.pallas_call(
        _main_kernel,
        grid=grid,
        in_specs=[
            pl.BlockSpec((bt, 2 * d), lambda i: (i, 0)),
            full((2 * d, 2 * d)),
            full((2 * d, s)),
            full((s, 2 * d)),
            full((1, 2 * d)),
        ],
        out_specs=pl.BlockSpec((bt, 2 * d), lambda i: (i, 0)),
        out_shape=jax.ShapeDtypeStruct((t, 2 * d), jnp.float32),
        scratch_shapes=[pltpu.VMEM((1, s), jnp.float32)],
    )(xc, wc, kc, vc, ginter)

    return o_c.reshape(b, l, d, 2)


# speedup vs baseline: 1.1695x; 1.1695x over previous
import functools

import jax
import jax.numpy as jnp
from jax.experimental import pallas as pl
from jax.experimental.pallas import tpu as pltpu

_TOPK = 8
_BLOCK_T = 256
_NEG = -1e30


def _main_kernel(xr_ref, xi_ref, wqr_ref, wqi_ref, ktr_ref, kti_ref,
                 vc_ref, ginter_ref, oc_ref, invk_ref):
    f32 = jnp.float32

    @pl.when(pl.program_id(0) == 0)
    def _():
        ktr0 = ktr_ref[...]
        kti0 = kti_ref[...]
        k_mag = jnp.sqrt(jnp.sum(ktr0 * ktr0, axis=0, keepdims=True)
                         + jnp.sum(kti0 * kti0, axis=0, keepdims=True) + 1e-8)
        invk_ref[...] = 1.0 / k_mag

    xr = xr_ref[...]
    xi = xi_ref[...]
    wqr = wqr_ref[...]
    wqi = wqi_ref[...]

    q_r = (jnp.dot(xr, wqr, preferred_element_type=f32)
           - jnp.dot(xi, wqi, preferred_element_type=f32))
    q_i = (jnp.dot(xr, wqi, preferred_element_type=f32)
           + jnp.dot(xi, wqr, preferred_element_type=f32))

    dot = (jnp.dot(q_r, ktr_ref[...], preferred_element_type=f32)
           + jnp.dot(q_i, kti_ref[...], preferred_element_type=f32))
    u = dot * invk_ref[...]

    q_mag = jnp.sqrt(jnp.sum(q_r * q_r, axis=1, keepdims=True)
                     + jnp.sum(q_i * q_i, axis=1, keepdims=True) + 1e-8)
    invq = 1.0 / q_mag

    m0 = jnp.max(u, axis=1, keepdims=True)
    oh = u == m0
    wd = jnp.where(oh, 1.0, 0.0)
    work = jnp.where(oh, _NEG, u)
    denom = jnp.ones_like(m0)
    for _ in range(_TOPK - 1):
        m = jnp.max(work, axis=1, keepdims=True)
        e = jnp.exp((m - m0) * invq)
        oh = work == m
        wd = jnp.where(oh, e, wd)
        work = jnp.where(oh, _NEG, work)
        denom = denom + e
    wd = wd * (1.0 / denom)

    out_c = jnp.dot(wd, vc_ref[...], preferred_element_type=f32)

    inv_rms = jax.lax.rsqrt(2.0 * jnp.mean(out_c * out_c, axis=1, keepdims=True)
                            + 1e-8)
    oc_ref[...] = out_c * inv_rms * ginter_ref[...]


@functools.partial(jax.jit, static_argnames=())
def kernel(x, keys, values, W_qr, W_qi, gamma):
    b, l, d, _ = x.shape
    s = keys.shape[0]
    t = b * l
    x_r = x[..., 0].reshape(t, d)
    x_i = x[..., 1].reshape(t, d)
    ktr = keys[..., 0].T
    kti = keys[..., 1].T
    vc = values.reshape(s, 2 * d)
    ginter = jnp.stack([gamma, gamma], axis=-1).reshape(1, 2 * d)

    bt = min(_BLOCK_T, t)
    grid = (t // bt,)
    full = lambda shape: pl.BlockSpec(shape, lambda i: (0, 0))

    o_c = pl.pallas_call(
        _main_kernel,
        grid=grid,
        in_specs=[
            pl.BlockSpec((bt, d), lambda i: (i, 0)),
            pl.BlockSpec((bt, d), lambda i: (i, 0)),
            full((d, d)), full((d, d)),
            full((d, s)), full((d, s)),
            full((s, 2 * d)),
            full((1, 2 * d)),
        ],
        out_specs=pl.BlockSpec((bt, 2 * d), lambda i: (i, 0)),
        out_shape=jax.ShapeDtypeStruct((t, 2 * d), jnp.float32),
        scratch_shapes=[pltpu.VMEM((1, s), jnp.float32)],
    )(x_r, x_i, W_qr, W_qi, ktr, kti, vc, ginter)

    return o_c.reshape(b, l, d, 2)


# speedup vs baseline: 1.3679x; 1.1696x over previous
import functools

import jax
import jax.numpy as jnp
from jax.experimental import pallas as pl
from jax.experimental.pallas import tpu as pltpu

_TOPK = 8
_BLOCK_T = 512
_NEG = -1e30


def _main_kernel(xr_ref, xi_ref, wqr_ref, wqi_ref, ktr_ref, kti_ref,
                 vr_ref, vi_ref, gamma_ref, or_ref, oi_ref, invk_ref):
    f32 = jnp.float32

    @pl.when(pl.program_id(0) == 0)
    def _():
        ktr0 = ktr_ref[...]
        kti0 = kti_ref[...]
        k_mag = jnp.sqrt(jnp.sum(ktr0 * ktr0, axis=0, keepdims=True)
                         + jnp.sum(kti0 * kti0, axis=0, keepdims=True) + 1e-8)
        invk_ref[...] = 1.0 / k_mag

    xr = xr_ref[...]
    xi = xi_ref[...]
    wqr = wqr_ref[...]
    wqi = wqi_ref[...]

    q_r = (jnp.dot(xr, wqr, preferred_element_type=f32)
           - jnp.dot(xi, wqi, preferred_element_type=f32))
    q_i = (jnp.dot(xr, wqi, preferred_element_type=f32)
           + jnp.dot(xi, wqr, preferred_element_type=f32))

    dot = (jnp.dot(q_r, ktr_ref[...], preferred_element_type=f32)
           + jnp.dot(q_i, kti_ref[...], preferred_element_type=f32))
    u = dot * invk_ref[...]

    q_mag = jnp.sqrt(jnp.sum(q_r * q_r, axis=1, keepdims=True)
                     + jnp.sum(q_i * q_i, axis=1, keepdims=True) + 1e-8)
    invq = 1.0 / q_mag

    m0 = jnp.max(u, axis=1, keepdims=True)
    oh = u == m0
    wd = jnp.where(oh, 1.0, 0.0)
    work = jnp.where(oh, _NEG, u)
    denom = jnp.ones_like(m0)
    for _ in range(_TOPK - 1):
        m = jnp.max(work, axis=1, keepdims=True)
        e = jnp.exp((m - m0) * invq)
        oh = work == m
        wd = jnp.where(oh, e, wd)
        work = jnp.where(oh, _NEG, work)
        denom = denom + e
    wd = wd * (1.0 / denom)

    out_r = jnp.dot(wd, vr_ref[...], preferred_element_type=f32)
    out_i = jnp.dot(wd, vi_ref[...], preferred_element_type=f32)

    mag2 = out_r * out_r + out_i * out_i
    inv_rms = jax.lax.rsqrt(jnp.mean(mag2, axis=1, keepdims=True) + 1e-8)
    gamma = gamma_ref[...]
    or_ref[...] = out_r * inv_rms * gamma
    oi_ref[...] = out_i * inv_rms * gamma


@functools.partial(jax.jit, static_argnames=())
def kernel(x, keys, values, W_qr, W_qi, gamma):
    b, l, d, _ = x.shape
    s = keys.shape[0]
    t = b * l
    x_r = x[..., 0].reshape(t, d)
    x_i = x[..., 1].reshape(t, d)
    ktr = keys[..., 0].T
    kti = keys[..., 1].T
    v_r = values[..., 0]
    v_i = values[..., 1]
    gamma2 = gamma.reshape(1, d)

    bt = min(_BLOCK_T, t)
    grid = (t // bt,)
    tok_spec = pl.BlockSpec((bt, d), lambda i: (i, 0))
    full = lambda shape: pl.BlockSpec(shape, lambda i: (0, 0))

    o_r, o_i = pl.pallas_call(
        _main_kernel,
        grid=grid,
        in_specs=[
            tok_spec, tok_spec,
            full((d, d)), full((d, d)),
            full((d, s)), full((d, s)),
            full((s, d)), full((s, d)),
            full((1, d)),
        ],
        out_specs=[tok_spec, tok_spec],
        out_shape=[
            jax.ShapeDtypeStruct((t, d), jnp.float32),
            jax.ShapeDtypeStruct((t, d), jnp.float32),
        ],
        scratch_shapes=[pltpu.VMEM((1, s), jnp.float32)],
    )(x_r, x_i, W_qr, W_qi, ktr, kti, v_r, v_i, gamma2)

    return jnp.stack([o_r, o_i], axis=-1).reshape(b, l, d, 2)
